# Initial kernel scaffold; baseline (speedup 1.0000x reference)
import jax
import jax.numpy as jnp
from jax.experimental import pallas as pl


def kernel(x, edge_index, raw_edge_attr, enc_W, enc_b, c1_Ws, c1_bs, c1_Wi, c1_Wo, c2_Ws, c2_bs, c2_Wi, c2_Wo, c3_Ws, c3_bs, c3_Wi, c3_Wo, m1_W, m1_b, m2_W, m2_b, m3_W, m3_b, m4_W, m4_b):
    raise NotImplementedError("write your pallas kernel here")


# speedup vs baseline: 6.8745x; 6.8745x over previous
import functools

import jax
import jax.numpy as jnp
from jax import lax
from jax.experimental import pallas as pl
from jax.experimental.pallas import tpu as pltpu
from jax.experimental.pallas import tpu_sc as plsc

N = 50000
E = 800000
H = 64
NC = 2
NS = 16
LANES = 16
HN = N // NC
ACC_ROWS = HN + 8
GW = 128
E_PAD = 802816
ZR = 128
BE = 4096
EG = E_PAD // BE
BN = 2000

_f32 = jnp.float32


def _leaky(v):
    return jnp.maximum(v, 0.2 * v)


def _dot(a, b):
    return jnp.dot(a, b, preferred_element_type=_f32,
                   precision=jax.lax.Precision.HIGHEST)


def _sc_mesh():
    return plsc.VectorSubcoreMesh(core_axis_name="core",
                                  subcore_axis_name="subcore")


_SC_PARAMS = pltpu.CompilerParams(use_tc_tiling_on_sc=False)


def _zero_fill(buf, rows):
    @pl.loop(0, rows)
    def _(i):
        for j in range(H // LANES):
            buf[pl.ds(i, 1), pl.ds(j * LANES, LANES)] = (
                jnp.zeros((1, LANES), _f32))


def _idx_transform(i_vmem, idxl, base):
    for j in range(GW // LANES):
        v = i_vmem[pl.ds(0, 1), pl.ds(j * LANES, LANES)]
        local = v - base
        local = jnp.where((local < 0) | (local >= HN), HN, local)
        idxl[pl.ds(0, 1), pl.ds(j * LANES, LANES)] = local


def _acc_writeout(acc, out_hbm, cid, sid):
    for k in range(13):
        c = k * NS + sid

        @pl.when(c < 200)
        def _():
            pltpu.sync_copy(acc.at[pl.ds(c * 125, 125)],
                            out_hbm.at[cid, pl.ds(c * 125, 125)])


def _sc_scatter_sum(vals, idx):
    @functools.partial(
        pl.kernel,
        out_type=jax.ShapeDtypeStruct((NC, HN, H), _f32),
        mesh=_sc_mesh(),
        compiler_params=_SC_PARAMS,
        scratch_types=[
            pltpu.VMEM_SHARED((ACC_ROWS, H), _f32),
            pltpu.VMEM((1, GW), jnp.int32),
            pltpu.VMEM((ZR, H), _f32),
        ],
    )
    def kern(vals_hbm, idx_hbm, out_hbm, acc, idxl, zbuf):
        cid = lax.axis_index("core")
        sid = lax.axis_index("subcore")
        base = cid * HN
        _zero_fill(zbuf, ZR)
        rows_per_tile = ACC_ROWS // NS
        for k in range(13):
            start = jnp.minimum(sid * rows_per_tile + k * ZR, ACC_ROWS - ZR)
            pltpu.sync_copy(zbuf, acc.at[pl.ds(start, ZR)])
        plsc.subcore_barrier()

        def body(v_vmem, i_vmem):
            _idx_transform(i_vmem, idxl, base)
            pltpu.sync_copy(v_vmem, acc.at[idxl.at[0]], add=True)

        pltpu.emit_pipeline(
            body,
            grid=(E_PAD // GW,),
            in_specs=[
                pl.BlockSpec((GW, H), lambda i: (i, 0)),
                pl.BlockSpec((1, GW), lambda i: (0, i)),
            ],
            out_specs=[],
            core_axis_name="subcore",
            dimension_semantics=(pltpu.PARALLEL,),
        )(vals_hbm, idx_hbm)
        plsc.subcore_barrier()
        _acc_writeout(acc, out_hbm, cid, sid)

    return kern(vals, idx).reshape(N, H)


def _sc_count(idx):
    @functools.partial(
        pl.kernel,
        out_type=jax.ShapeDtypeStruct((NC, HN, H), _f32),
        mesh=_sc_mesh(),
        compiler_params=_SC_PARAMS,
        scratch_types=[
            pltpu.VMEM_SHARED((ACC_ROWS, H), _f32),
            pltpu.VMEM((1, GW), jnp.int32),
            pltpu.VMEM((ZR, H), _f32),
            pltpu.VMEM((GW, H), _f32),
        ],
    )
    def kern(idx_hbm, out_hbm, acc, idxl, zbuf, ones_v):
        cid = lax.axis_index("core")
        sid = lax.axis_index("subcore")
        base = cid * HN
        _zero_fill(zbuf, ZR)

        @pl.loop(0, GW)
        def _(i):
            for j in range(H // LANES):
                ones_v[pl.ds(i, 1), pl.ds(j * LANES, LANES)] = (
                    jnp.ones((1, LANES), _f32))

        rows_per_tile = ACC_ROWS // NS
        for k in range(13):
            start = jnp.minimum(sid * rows_per_tile + k * ZR, ACC_ROWS - ZR)
            pltpu.sync_copy(zbuf, acc.at[pl.ds(start, ZR)])
        plsc.subcore_barrier()

        def body(i_vmem):
            _idx_transform(i_vmem, idxl, base)
            pltpu.sync_copy(ones_v, acc.at[idxl.at[0]], add=True)

        pltpu.emit_pipeline(
            body,
            grid=(E_PAD // GW,),
            in_specs=[pl.BlockSpec((1, GW), lambda i: (0, i))],
            out_specs=[],
            core_axis_name="subcore",
            dimension_semantics=(pltpu.PARALLEL,),
        )(idx_hbm)
        plsc.subcore_barrier()
        _acc_writeout(acc, out_hbm, cid, sid)

    return kern(idx).reshape(N, H)


def _sc_gather2(ta, tb, src_g, dst_g):
    @functools.partial(
        pl.kernel,
        out_type=[jax.ShapeDtypeStruct((E_PAD, H), _f32),
                  jax.ShapeDtypeStruct((E_PAD, H), _f32)],
        mesh=_sc_mesh(),
        compiler_params=_SC_PARAMS,
    )
    def kern(ta_hbm, tb_hbm, si_hbm, di_hbm, ga_hbm, gb_hbm):
        def body(si_vmem, di_vmem, ga_vmem, gb_vmem):
            pltpu.sync_copy(ta_hbm.at[si_vmem.at[0]], ga_vmem)
            pltpu.sync_copy(tb_hbm.at[di_vmem.at[0]], gb_vmem)

        pltpu.emit_pipeline(
            body,
            grid=(E_PAD // GW,),
            in_specs=[pl.BlockSpec((1, GW), lambda i: (0, i)),
                      pl.BlockSpec((1, GW), lambda i: (0, i))],
            out_specs=[pl.BlockSpec((GW, H), lambda i: (i, 0)),
                       pl.BlockSpec((GW, H), lambda i: (i, 0))],
            core_axis_name=("core", "subcore"),
            dimension_semantics=(pltpu.PARALLEL,),
        )(si_hbm, di_hbm, ga_hbm, gb_hbm)

    return kern(ta, tb, src_g, dst_g)


def _w_spec(r, c):
    return pl.BlockSpec((r, c), lambda i: (0, 0))


def _tc_enc_node(x_pad, w_src, w_dst):
    def body(x_ref, ws_ref, wd_ref, xa_ref, xb_ref):
        xv = x_ref[...]
        xa_ref[...] = _dot(xv, ws_ref[...])
        xb_ref[...] = _dot(xv, wd_ref[...])

    return pl.pallas_call(
        body,
        grid=(N // BN,),
        in_specs=[pl.BlockSpec((BN, 16), lambda i: (i, 0)),
                  _w_spec(16, H), _w_spec(16, H)],
        out_specs=[pl.BlockSpec((BN, H), lambda i: (i, 0)),
                   pl.BlockSpec((BN, H), lambda i: (i, 0))],
        out_shape=[jax.ShapeDtypeStruct((N, H), _f32),
                   jax.ShapeDtypeStruct((N, H), _f32)],
    )(x_pad, w_src, w_dst)


def _tc_node(a, b, cnt_d, cnt_s, wi, wo):
    def body(a_ref, b_ref, cd_ref, cs_ref, wi_ref, wo_ref, ta_ref, tb_ref):
        cd = jnp.maximum(cd_ref[:, 0:1], 1.0)
        cs = jnp.maximum(cs_ref[:, 0:1], 1.0)
        ta_ref[...] = _dot(a_ref[...] / cd, wi_ref[...])
        tb_ref[...] = _dot(b_ref[...] / cs, wo_ref[...])

    return pl.pallas_call(
        body,
        grid=(N // BN,),
        in_specs=[pl.BlockSpec((BN, H), lambda i: (i, 0)),
                  pl.BlockSpec((BN, H), lambda i: (i, 0)),
                  pl.BlockSpec((BN, H), lambda i: (i, 0)),
                  pl.BlockSpec((BN, H), lambda i: (i, 0)),
                  _w_spec(H, H), _w_spec(H, H)],
        out_specs=[pl.BlockSpec((BN, H), lambda i: (i, 0)),
                   pl.BlockSpec((BN, H), lambda i: (i, 0))],
        out_shape=[jax.ShapeDtypeStruct((N, H), _f32),
                   jax.ShapeDtypeStruct((N, H), _f32)],
    )(a, b, cnt_d, cnt_s, wi, wo)


def _tc_combine_enc(e_pad, ga, gb, w_e_pad, enc_b, ws1, bs1):
    def body(e_ref, ga_ref, gb_ref, we_ref, eb_ref, w_ref, b_ref,
             h_ref, s_ref):
        hv = _leaky(_dot(e_ref[...], we_ref[...]) + eb_ref[...]
                    + ga_ref[...] + gb_ref[...])
        h_ref[...] = hv
        s_ref[...] = _dot(hv, w_ref[...]) + b_ref[...]

    return pl.pallas_call(
        body,
        grid=(EG,),
        in_specs=[pl.BlockSpec((BE, 8), lambda i: (i, 0)),
                  pl.BlockSpec((BE, H), lambda i: (i, 0)),
                  pl.BlockSpec((BE, H), lambda i: (i, 0)),
                  _w_spec(8, H), _w_spec(1, H), _w_spec(H, H), _w_spec(1, H)],
        out_specs=[pl.BlockSpec((BE, H), lambda i: (i, 0)),
                   pl.BlockSpec((BE, H), lambda i: (i, 0))],
        out_shape=[jax.ShapeDtypeStruct((E_PAD, H), _f32),
                   jax.ShapeDtypeStruct((E_PAD, H), _f32)],
    )(e_pad, ga, gb, w_e_pad, enc_b, ws1, bs1)


def _tc_combine_conv(s, ga, gb, w_next, b_next):
    def body(s_ref, ga_ref, gb_ref, w_ref, b_ref, h_ref, so_ref):
        hv = _leaky(s_ref[...] + ga_ref[...] + gb_ref[...])
        h_ref[...] = hv
        so_ref[...] = _dot(hv, w_ref[...]) + b_ref[...]

    return pl.pallas_call(
        body,
        grid=(EG,),
        in_specs=[pl.BlockSpec((BE, H), lambda i: (i, 0)),
                  pl.BlockSpec((BE, H), lambda i: (i, 0)),
                  pl.BlockSpec((BE, H), lambda i: (i, 0)),
                  _w_spec(H, H), _w_spec(1, H)],
        out_specs=[pl.BlockSpec((BE, H), lambda i: (i, 0)),
                   pl.BlockSpec((BE, H), lambda i: (i, 0))],
        out_shape=[jax.ShapeDtypeStruct((E_PAD, H), _f32),
                   jax.ShapeDtypeStruct((E_PAD, H), _f32)],
    )(s, ga, gb, w_next, b_next)


def _tc_combine_mlp(s, ga, gb, m1w, m1b, m2w, m2b, m3w, m3b, m4w, m4b):
    def body(s_ref, ga_ref, gb_ref, w1_ref, b1_ref, w2_ref, b2_ref,
             w3_ref, b3_ref, w4_ref, b4_ref, o_ref):
        hv = _leaky(s_ref[...] + ga_ref[...] + gb_ref[...])
        hv = _leaky(_dot(hv, w1_ref[...]) + b1_ref[...])
        hv = _leaky(_dot(hv, w2_ref[...]) + b2_ref[...])
        hv = _leaky(_dot(hv, w3_ref[...]) + b3_ref[...])
        o_ref[...] = jnp.sum(hv * w4_ref[...], axis=1) + b4_ref[0, 0]

    return pl.pallas_call(
        body,
        grid=(EG,),
        in_specs=[pl.BlockSpec((BE, H), lambda i: (i, 0)),
                  pl.BlockSpec((BE, H), lambda i: (i, 0)),
                  pl.BlockSpec((BE, H), lambda i: (i, 0)),
                  _w_spec(H, H), _w_spec(1, H),
                  _w_spec(H, H), _w_spec(1, H),
                  _w_spec(H, 32), _w_spec(1, 32),
                  _w_spec(1, 32),
                  pl.BlockSpec((1, 1), lambda i: (0, 0),
                               memory_space=pltpu.SMEM)],
        out_specs=pl.BlockSpec((BE,), lambda i: (i,)),
        out_shape=jax.ShapeDtypeStruct((E,), _f32),
    )(s, ga, gb, m1w, m1b, m2w, m2b, m3w, m3b, m4w, m4b)


def kernel(x, edge_index, raw_edge_attr, enc_W, enc_b,
           c1_Ws, c1_bs, c1_Wi, c1_Wo,
           c2_Ws, c2_bs, c2_Wi, c2_Wo,
           c3_Ws, c3_bs, c3_Wi, c3_Wo,
           m1_W, m1_b, m2_W, m2_b, m3_W, m3_b, m4_W, m4_b):
    src = edge_index[0].astype(jnp.int32)
    dst = edge_index[1].astype(jnp.int32)
    npad = E_PAD - E
    pad_g = jnp.zeros((npad,), jnp.int32)
    pad_s = jnp.full((npad,), N, jnp.int32)
    src_g = jnp.concatenate([src, pad_g]).reshape(1, E_PAD)
    dst_g = jnp.concatenate([dst, pad_g]).reshape(1, E_PAD)
    src_s = jnp.concatenate([src, pad_s]).reshape(1, E_PAD)
    dst_s = jnp.concatenate([dst, pad_s]).reshape(1, E_PAD)

    x_pad = jnp.pad(x, ((0, 0), (0, 3)))
    w_src = jnp.pad(enc_W[0:13], ((0, 3), (0, 0)))
    w_dst = jnp.pad(enc_W[13:26], ((0, 3), (0, 0)))
    e_pad = jnp.pad(raw_edge_attr, ((0, 0), (0, 7)))
    w_e_pad = jnp.pad(enc_W[26:27], ((0, 7), (0, 0)))

    cnt_d = _sc_count(dst_s)
    cnt_s = _sc_count(src_s)

    xa, xb = _tc_enc_node(x_pad, w_src, w_dst)
    ga, gb = _sc_gather2(xa, xb, src_g, dst_g)
    h, s = _tc_combine_enc(e_pad, ga, gb, w_e_pad, enc_b.reshape(1, H),
                           c1_Ws, c1_bs.reshape(1, H))

    convs = [(c1_Wi, c1_Wo, c2_Ws, c2_bs), (c2_Wi, c2_Wo, c3_Ws, c3_bs)]
    for wi, wo, ws_n, bs_n in convs:
        a = _sc_scatter_sum(h, dst_s)
        b = _sc_scatter_sum(h, src_s)
        ta, tb = _tc_node(a, b, cnt_d, cnt_s, wi, wo)
        ga, gb = _sc_gather2(ta, tb, src_g, dst_g)
        h, s = _tc_combine_conv(s, ga, gb, ws_n, bs_n.reshape(1, H))

    a = _sc_scatter_sum(h, dst_s)
    b = _sc_scatter_sum(h, src_s)
    ta, tb = _tc_node(a, b, cnt_d, cnt_s, c3_Wi, c3_Wo)
    ga, gb = _sc_gather2(ta, tb, src_g, dst_g)
    out = _tc_combine_mlp(s, ga, gb,
                          m1_W, m1_b.reshape(1, H),
                          m2_W, m2_b.reshape(1, H),
                          m3_W, m3_b.reshape(1, 32),
                          m4_W.reshape(1, 32), m4_b.reshape(1, 1))
    return out
